# baseline (speedup 1.0000x reference)
import functools

import jax
import jax.numpy as jnp
from jax import lax
from jax.experimental import pallas as pl
from jax.experimental.pallas import tpu as pltpu
from jax.experimental.pallas import tpu_sc as plsc

BATCH = 4096
TL = 20
VOCAB = 100000
DIM = 768
NTOK = BATCH * TL
LANES = 16

NUM_CORES = 2
NUM_SUBCORES = 16
NW = NUM_CORES * NUM_SUBCORES

NSPLIT = 4
CH_ROWS = NTOK // NSPLIT
B_PER_W = CH_ROWS // NW
CHUNK = 64
PK = CHUNK // 2
NSTREAM = B_PER_W // CHUNK

ROWS_BLK = 1024
WORD_BLK = ROWS_BLK // 2
BLK_PER_CH = CH_ROWS // ROWS_BLK


def _sc_gather_pack(table, idx_c):
    mesh = plsc.VectorSubcoreMesh(
        core_axis_name="c", subcore_axis_name="s",
        num_cores=NUM_CORES, num_subcores=NUM_SUBCORES)

    @functools.partial(
        pl.kernel,
        out_type=jax.ShapeDtypeStruct((CH_ROWS // 2, DIM), jnp.int32),
        mesh=mesh,
        compiler_params=pltpu.CompilerParams(use_tc_tiling_on_sc=False,
                                             needs_layout_passes=False),
        scratch_types=[
            pltpu.VMEM((B_PER_W,), jnp.int32),
            pltpu.VMEM((CHUNK, DIM), jnp.float32),
            pltpu.VMEM((CHUNK, DIM), jnp.float32),
            pltpu.VMEM((PK, DIM), jnp.int32),
            pltpu.SemaphoreType.DMA,
        ],
    )
    def gather_kernel(table_hbm, idx_hbm, out_hbm, idx_v, rows_a, rows_b,
                      pk_v, sem):
        wid = lax.axis_index("s") * NUM_CORES + lax.axis_index("c")
        base = wid * B_PER_W
        pltpu.sync_copy(idx_hbm.at[pl.ds(base, B_PER_W)], idx_v)
        bufs = (rows_a, rows_b)
        pltpu.async_copy(
            table_hbm.at[idx_v.at[pl.ds(0, CHUNK)]], rows_a, sem)
        for c in range(NSTREAM):
            cur = bufs[c % 2]
            pltpu.make_async_copy(
                table_hbm.at[pl.ds(0, CHUNK)], cur, sem).wait()
            if c + 1 < NSTREAM:
                pltpu.async_copy(
                    table_hbm.at[idx_v.at[pl.ds((c + 1) * CHUNK, CHUNK)]],
                    bufs[(c + 1) % 2], sem)

            def pack_row(i, carry, cur=cur):
                for l in range(DIM // LANES):
                    a = cur[2 * i, pl.ds(l * LANES, LANES)]
                    b = cur[2 * i + 1, pl.ds(l * LANES, LANES)]
                    au = plsc.bitcast(a, jnp.uint32)
                    bu = plsc.bitcast(b, jnp.uint32)
                    ar = (au + jnp.uint32(0x7FFF) + ((au >> 16) & jnp.uint32(1))) >> 16
                    br = (bu + jnp.uint32(0x7FFF) + ((bu >> 16) & jnp.uint32(1))) & jnp.uint32(0xFFFF0000)
                    pk_v[i, pl.ds(l * LANES, LANES)] = plsc.bitcast(
                        ar | br, jnp.int32)
                return carry

            lax.fori_loop(0, PK, pack_row, 0)
            pltpu.sync_copy(
                pk_v, out_hbm.at[pl.ds(base // 2 + c * PK, PK)])

    return gather_kernel(table, idx_c)


def _mm_body(x_ref, w_ref, b_ref, o_ref):
    x32 = x_ref[...]
    lo = lax.bitcast_convert_type(
        x32 << 16, jnp.float32).astype(jnp.bfloat16)
    hi = lax.bitcast_convert_type(
        x32 & jnp.int32(-65536), jnp.float32).astype(jnp.bfloat16)
    w = w_ref[...].astype(jnp.bfloat16)
    acc_lo = lax.dot_general(lo, w, (((1,), (1,)), ((), ())),
                             preferred_element_type=jnp.float32)
    acc_hi = lax.dot_general(hi, w, (((1,), (1,)), ((), ())),
                             preferred_element_type=jnp.float32)
    bias = b_ref[...]
    o_ref[0:WORD_BLK, :] = acc_lo + bias
    o_ref[WORD_BLK:ROWS_BLK, :] = acc_hi + bias


def _mm_body_alias(x_ref, w_ref, b_ref, prev_ref, o_ref):
    del prev_ref
    _mm_body(x_ref, w_ref, b_ref, o_ref)


def _tc_transform_chunk(x, W2, b2, prev, c):
    out_map = functools.partial(lambda c_, i: (c_ * BLK_PER_CH + i, 0), c)
    x_spec = pl.BlockSpec((WORD_BLK, DIM), lambda i: (i, 0))
    w_spec = pl.BlockSpec((DIM, DIM), lambda i: (0, 0))
    b_spec = pl.BlockSpec((1, DIM), lambda i: (0, 0))
    if prev is None:
        return pl.pallas_call(
            _mm_body,
            grid=(BLK_PER_CH,),
            in_specs=[x_spec, w_spec, b_spec],
            out_specs=pl.BlockSpec((ROWS_BLK, DIM), out_map),
            out_shape=jax.ShapeDtypeStruct((NTOK, DIM), jnp.float32),
        )(x, W2, b2)
    return pl.pallas_call(
        _mm_body_alias,
        grid=(BLK_PER_CH,),
        in_specs=[x_spec, w_spec, b_spec,
                  pl.BlockSpec(memory_space=pl.ANY)],
        out_specs=pl.BlockSpec((ROWS_BLK, DIM), out_map),
        out_shape=jax.ShapeDtypeStruct((NTOK, DIM), jnp.float32),
        input_output_aliases={3: 0},
    )(x, W2, b2, prev)


def kernel(token_ids, joint_embed, W, b):
    idx = token_ids.T.reshape(-1)
    idx = idx.reshape(NTOK // ROWS_BLK, 2, WORD_BLK).transpose(0, 2, 1)
    idx = idx.reshape(-1)
    b2 = b.reshape(1, DIM)
    packed = [
        _sc_gather_pack(joint_embed,
                        lax.slice(idx, (c * CH_ROWS,), ((c + 1) * CH_ROWS,)))
        for c in range(NSPLIT)
    ]
    out2d = None
    for c in range(NSPLIT):
        out2d = _tc_transform_chunk(packed[c], W, b2, out2d, c)
    return out2d.reshape(TL, BATCH, DIM).transpose(1, 0, 2)


# speedup vs baseline: 1.1653x; 1.1653x over previous
import functools

import jax
import jax.numpy as jnp
from jax import lax
from jax.experimental import pallas as pl
from jax.experimental.pallas import tpu as pltpu
from jax.experimental.pallas import tpu_sc as plsc

BATCH = 4096
TL = 20
VOCAB = 100000
DIM = 768
NTOK = BATCH * TL
LANES = 16

NUM_CORES = 2
NUM_SUBCORES = 16
NW = NUM_CORES * NUM_SUBCORES

NSPLIT = 4
CH_ROWS = NTOK // NSPLIT
B_PER_W = CH_ROWS // NW
CHUNK = 64
PK = CHUNK // 2
NSTREAM = B_PER_W // CHUNK

ROWS_BLK = 1024
WORD_BLK = ROWS_BLK // 2
BLK_PER_CH = CH_ROWS // ROWS_BLK


def _sc_gather_pack(table, idx_c):
    mesh = plsc.VectorSubcoreMesh(
        core_axis_name="c", subcore_axis_name="s",
        num_cores=NUM_CORES, num_subcores=NUM_SUBCORES)

    @functools.partial(
        pl.kernel,
        out_type=jax.ShapeDtypeStruct((CH_ROWS // 2, DIM), jnp.int32),
        mesh=mesh,
        compiler_params=pltpu.CompilerParams(use_tc_tiling_on_sc=False,
                                             needs_layout_passes=False),
        scratch_types=[
            pltpu.VMEM((B_PER_W,), jnp.int32),
            pltpu.VMEM((CHUNK, DIM), jnp.float32),
            pltpu.VMEM((CHUNK, DIM), jnp.float32),
            pltpu.VMEM((PK, DIM), jnp.int32),
            pltpu.SemaphoreType.DMA,
        ],
    )
    def gather_kernel(table_hbm, idx_hbm, out_hbm, idx_v, rows_a, rows_b,
                      pk_v, sem):
        wid = lax.axis_index("s") * NUM_CORES + lax.axis_index("c")
        base = wid * B_PER_W
        pltpu.sync_copy(idx_hbm.at[pl.ds(base, B_PER_W)], idx_v)
        bufs = (rows_a, rows_b)
        pltpu.async_copy(
            table_hbm.at[idx_v.at[pl.ds(0, CHUNK)]], rows_a, sem)
        for c in range(NSTREAM):
            cur = bufs[c % 2]
            pltpu.make_async_copy(
                table_hbm.at[pl.ds(0, CHUNK)], cur, sem).wait()
            if c + 1 < NSTREAM:
                pltpu.async_copy(
                    table_hbm.at[idx_v.at[pl.ds((c + 1) * CHUNK, CHUNK)]],
                    bufs[(c + 1) % 2], sem)

            def pack_row(i, carry, cur=cur):
                for l in range(DIM // LANES):
                    a = cur[2 * i, pl.ds(l * LANES, LANES)]
                    b = cur[2 * i + 1, pl.ds(l * LANES, LANES)]
                    au = plsc.bitcast(a, jnp.uint32)
                    bu = plsc.bitcast(b, jnp.uint32)
                    pk_v[i, pl.ds(l * LANES, LANES)] = plsc.bitcast(
                        (au >> 16) | (bu & jnp.uint32(0xFFFF0000)), jnp.int32)
                return carry

            lax.fori_loop(0, PK, pack_row, 0)
            pltpu.sync_copy(
                pk_v, out_hbm.at[pl.ds(base // 2 + c * PK, PK)])

    return gather_kernel(table, idx_c)


def _mm_body(x_ref, w_ref, b_ref, o_ref):
    x32 = x_ref[...]
    lo = lax.bitcast_convert_type(
        x32 << 16, jnp.float32).astype(jnp.bfloat16)
    hi = lax.bitcast_convert_type(
        x32 & jnp.int32(-65536), jnp.float32).astype(jnp.bfloat16)
    w = w_ref[...].astype(jnp.bfloat16)
    acc_lo = lax.dot_general(lo, w, (((1,), (1,)), ((), ())),
                             preferred_element_type=jnp.float32)
    acc_hi = lax.dot_general(hi, w, (((1,), (1,)), ((), ())),
                             preferred_element_type=jnp.float32)
    bias = b_ref[...]
    o_ref[0:WORD_BLK, :] = acc_lo + bias
    o_ref[WORD_BLK:ROWS_BLK, :] = acc_hi + bias


def _mm_body_alias(x_ref, w_ref, b_ref, prev_ref, o_ref):
    del prev_ref
    _mm_body(x_ref, w_ref, b_ref, o_ref)


def _tc_transform_chunk(x, W2, b2, prev, c):
    out_map = functools.partial(lambda c_, i: (c_ * BLK_PER_CH + i, 0), c)
    x_spec = pl.BlockSpec((WORD_BLK, DIM), lambda i: (i, 0))
    w_spec = pl.BlockSpec((DIM, DIM), lambda i: (0, 0))
    b_spec = pl.BlockSpec((1, DIM), lambda i: (0, 0))
    if prev is None:
        return pl.pallas_call(
            _mm_body,
            grid=(BLK_PER_CH,),
            in_specs=[x_spec, w_spec, b_spec],
            out_specs=pl.BlockSpec((ROWS_BLK, DIM), out_map),
            out_shape=jax.ShapeDtypeStruct((NTOK, DIM), jnp.float32),
        )(x, W2, b2)
    return pl.pallas_call(
        _mm_body_alias,
        grid=(BLK_PER_CH,),
        in_specs=[x_spec, w_spec, b_spec,
                  pl.BlockSpec(memory_space=pl.ANY)],
        out_specs=pl.BlockSpec((ROWS_BLK, DIM), out_map),
        out_shape=jax.ShapeDtypeStruct((NTOK, DIM), jnp.float32),
        input_output_aliases={3: 0},
    )(x, W2, b2, prev)


def kernel(token_ids, joint_embed, W, b):
    idx = token_ids.T.reshape(-1)
    idx = idx.reshape(NTOK // ROWS_BLK, 2, WORD_BLK).transpose(0, 2, 1)
    idx = idx.reshape(-1)
    b2 = b.reshape(1, DIM)
    packed = [
        _sc_gather_pack(joint_embed,
                        lax.slice(idx, (c * CH_ROWS,), ((c + 1) * CH_ROWS,)))
        for c in range(NSPLIT)
    ]
    out2d = None
    for c in range(NSPLIT):
        out2d = _tc_transform_chunk(packed[c], W, b2, out2d, c)
    return out2d.reshape(TL, BATCH, DIM).transpose(1, 0, 2)


# speedup vs baseline: 1.3910x; 1.1936x over previous
import functools

import jax
import jax.numpy as jnp
from jax import lax
from jax.experimental import pallas as pl
from jax.experimental.pallas import tpu as pltpu
from jax.experimental.pallas import tpu_sc as plsc

BATCH = 4096
TL = 20
VOCAB = 100000
DIM = 768
NTOK = BATCH * TL
LANES = 16

NUM_CORES = 2
NUM_SUBCORES = 16
NW = NUM_CORES * NUM_SUBCORES

NSPLIT = 4
CH_ROWS = NTOK // NSPLIT
B_PER_W = CH_ROWS // NW
CHUNK = 64
PK = CHUNK // 2
NSTREAM = B_PER_W // CHUNK

ROWS_BLK = 1024
WORD_BLK = ROWS_BLK // 2
BLK_PER_CH = CH_ROWS // ROWS_BLK


def _sc_gather_pack(table, idx_c):
    mesh = plsc.VectorSubcoreMesh(
        core_axis_name="c", subcore_axis_name="s",
        num_cores=NUM_CORES, num_subcores=NUM_SUBCORES)

    @functools.partial(
        pl.kernel,
        out_type=jax.ShapeDtypeStruct((CH_ROWS // 2, DIM), jnp.int32),
        mesh=mesh,
        compiler_params=pltpu.CompilerParams(use_tc_tiling_on_sc=False,
                                             needs_layout_passes=False),
        scratch_types=[
            pltpu.VMEM((B_PER_W,), jnp.int32),
            pltpu.VMEM((CHUNK, DIM), jnp.float32),
            pltpu.VMEM((CHUNK, DIM), jnp.float32),
            pltpu.VMEM((PK, DIM), jnp.int32),
            pltpu.SemaphoreType.DMA,
        ],
    )
    def gather_kernel(table_hbm, idx_hbm, out_hbm, idx_v, rows_a, rows_b,
                      pk_v, sem):
        wid = lax.axis_index("s") * NUM_CORES + lax.axis_index("c")
        base = wid * B_PER_W
        pltpu.sync_copy(idx_hbm.at[pl.ds(base, B_PER_W)], idx_v)
        bufs = (rows_a, rows_b)
        pltpu.async_copy(
            table_hbm.at[idx_v.at[pl.ds(0, CHUNK)]], rows_a, sem)
        for c in range(NSTREAM):
            cur = bufs[c % 2]
            pltpu.make_async_copy(
                table_hbm.at[pl.ds(0, CHUNK)], cur, sem).wait()
            if c + 1 < NSTREAM:
                pltpu.async_copy(
                    table_hbm.at[idx_v.at[pl.ds((c + 1) * CHUNK, CHUNK)]],
                    bufs[(c + 1) % 2], sem)

            @plsc.parallel_loop(0, PK, unroll=4)
            def pack_row(i, cur=cur):
                for l in range(DIM // LANES):
                    a = cur[2 * i, pl.ds(l * LANES, LANES)]
                    b = cur[2 * i + 1, pl.ds(l * LANES, LANES)]
                    au = plsc.bitcast(a, jnp.uint32)
                    bu = plsc.bitcast(b, jnp.uint32)
                    pk_v[i, pl.ds(l * LANES, LANES)] = plsc.bitcast(
                        (au >> 16) | (bu & jnp.uint32(0xFFFF0000)), jnp.int32)
            pltpu.sync_copy(
                pk_v, out_hbm.at[pl.ds(base // 2 + c * PK, PK)])

    return gather_kernel(table, idx_c)


def _mm_body(x_ref, w_ref, b_ref, o_ref):
    x32 = x_ref[...]
    lo = lax.bitcast_convert_type(
        x32 << 16, jnp.float32).astype(jnp.bfloat16)
    hi = lax.bitcast_convert_type(
        x32 & jnp.int32(-65536), jnp.float32).astype(jnp.bfloat16)
    w = w_ref[...].astype(jnp.bfloat16)
    acc_lo = lax.dot_general(lo, w, (((1,), (1,)), ((), ())),
                             preferred_element_type=jnp.float32)
    acc_hi = lax.dot_general(hi, w, (((1,), (1,)), ((), ())),
                             preferred_element_type=jnp.float32)
    bias = b_ref[...]
    o_ref[0:WORD_BLK, :] = acc_lo + bias
    o_ref[WORD_BLK:ROWS_BLK, :] = acc_hi + bias


def _mm_body_alias(x_ref, w_ref, b_ref, prev_ref, o_ref):
    del prev_ref
    _mm_body(x_ref, w_ref, b_ref, o_ref)


def _tc_transform_chunk(x, W2, b2, prev, c):
    out_map = functools.partial(lambda c_, i: (c_ * BLK_PER_CH + i, 0), c)
    x_spec = pl.BlockSpec((WORD_BLK, DIM), lambda i: (i, 0))
    w_spec = pl.BlockSpec((DIM, DIM), lambda i: (0, 0))
    b_spec = pl.BlockSpec((1, DIM), lambda i: (0, 0))
    if prev is None:
        return pl.pallas_call(
            _mm_body,
            grid=(BLK_PER_CH,),
            in_specs=[x_spec, w_spec, b_spec],
            out_specs=pl.BlockSpec((ROWS_BLK, DIM), out_map),
            out_shape=jax.ShapeDtypeStruct((NTOK, DIM), jnp.float32),
        )(x, W2, b2)
    return pl.pallas_call(
        _mm_body_alias,
        grid=(BLK_PER_CH,),
        in_specs=[x_spec, w_spec, b_spec,
                  pl.BlockSpec(memory_space=pl.ANY)],
        out_specs=pl.BlockSpec((ROWS_BLK, DIM), out_map),
        out_shape=jax.ShapeDtypeStruct((NTOK, DIM), jnp.float32),
        input_output_aliases={3: 0},
    )(x, W2, b2, prev)


def kernel(token_ids, joint_embed, W, b):
    idx = token_ids.T.reshape(-1)
    idx = idx.reshape(NTOK // ROWS_BLK, 2, WORD_BLK).transpose(0, 2, 1)
    idx = idx.reshape(-1)
    b2 = b.reshape(1, DIM)
    packed = [
        _sc_gather_pack(joint_embed,
                        lax.slice(idx, (c * CH_ROWS,), ((c + 1) * CH_ROWS,)))
        for c in range(NSPLIT)
    ]
    out2d = None
    for c in range(NSPLIT):
        out2d = _tc_transform_chunk(packed[c], W, b2, out2d, c)
    return out2d.reshape(TL, BATCH, DIM).transpose(1, 0, 2)


# speedup vs baseline: 2.8617x; 2.0574x over previous
import functools

import jax
import jax.numpy as jnp
from jax import lax
from jax.experimental import pallas as pl
from jax.experimental.pallas import tpu as pltpu
from jax.experimental.pallas import tpu_sc as plsc

BATCH = 4096
TL = 20
VOCAB = 100000
DIM = 768
NTOK = BATCH * TL

NUM_CORES = 2
NUM_SUBCORES = 16
NW = NUM_CORES * NUM_SUBCORES
CHUNK = 128
UNIT = NW * CHUNK

ROWS_BLK = 1024

SPLITS = (2, 5, 5, 5, 3)


def _sc_gather_chunk(table, idx_c, units):
    n_rows = units * UNIT
    b_per_w = n_rows // NW
    mesh = plsc.VectorSubcoreMesh(
        core_axis_name="c", subcore_axis_name="s",
        num_cores=NUM_CORES, num_subcores=NUM_SUBCORES)

    @functools.partial(
        pl.kernel,
        out_type=jax.ShapeDtypeStruct((n_rows, DIM), jnp.float32),
        mesh=mesh,
        compiler_params=pltpu.CompilerParams(use_tc_tiling_on_sc=True),
        scratch_types=[
            pltpu.VMEM((b_per_w,), jnp.int32),
            pltpu.VMEM((CHUNK, DIM), jnp.float32),
            pltpu.SemaphoreType.DMA,
        ],
    )
    def gather_kernel(table_hbm, idx_hbm, out_hbm, idx_v, rows_v, sem):
        wid = lax.axis_index("s") * NUM_CORES + lax.axis_index("c")
        base = wid * b_per_w
        pltpu.sync_copy(idx_hbm.at[pl.ds(base, b_per_w)], idx_v)
        for c in range(units):
            pltpu.async_copy(
                table_hbm.at[idx_v.at[pl.ds(c * CHUNK, CHUNK)]],
                rows_v, sem).wait()
            pltpu.sync_copy(
                rows_v, out_hbm.at[pl.ds(base + c * CHUNK, CHUNK)])

    return gather_kernel(table, idx_c)


def _mm_body(x_ref, w_ref, b_ref, o_ref):
    x = x_ref[...].astype(jnp.bfloat16)
    w = w_ref[...].astype(jnp.bfloat16)
    acc = lax.dot_general(x, w, (((1,), (1,)), ((), ())),
                          preferred_element_type=jnp.float32)
    o_ref[...] = acc + b_ref[...]


def _mm_body_alias(x_ref, w_ref, b_ref, prev_ref, o_ref):
    del prev_ref
    _mm_body(x_ref, w_ref, b_ref, o_ref)


def _tc_transform_chunk(x, W2, b2, prev, row0, units):
    blocks = units * UNIT // ROWS_BLK
    blk0 = row0 // ROWS_BLK
    out_map = functools.partial(lambda b0, i: (b0 + i, 0), blk0)
    x_spec = pl.BlockSpec((ROWS_BLK, DIM), lambda i: (i, 0))
    w_spec = pl.BlockSpec((DIM, DIM), lambda i: (0, 0))
    b_spec = pl.BlockSpec((1, DIM), lambda i: (0, 0))
    if prev is None:
        return pl.pallas_call(
            _mm_body,
            grid=(blocks,),
            in_specs=[x_spec, w_spec, b_spec],
            out_specs=pl.BlockSpec((ROWS_BLK, DIM), out_map),
            out_shape=jax.ShapeDtypeStruct((NTOK, DIM), jnp.float32),
        )(x, W2, b2)
    return pl.pallas_call(
        _mm_body_alias,
        grid=(blocks,),
        in_specs=[x_spec, w_spec, b_spec,
                  pl.BlockSpec(memory_space=pl.ANY)],
        out_specs=pl.BlockSpec((ROWS_BLK, DIM), out_map),
        out_shape=jax.ShapeDtypeStruct((NTOK, DIM), jnp.float32),
        input_output_aliases={3: 0},
    )(x, W2, b2, prev)


def kernel(token_ids, joint_embed, W, b):
    idx = token_ids.T.reshape(-1)
    b2 = b.reshape(1, DIM)
    offs = [0]
    for u in SPLITS:
        offs.append(offs[-1] + u * UNIT)
    embeds = [
        _sc_gather_chunk(joint_embed,
                         lax.slice(idx, (offs[c],), (offs[c + 1],)),
                         SPLITS[c])
        for c in range(len(SPLITS))
    ]
    out2d = None
    for c in range(len(SPLITS)):
        out2d = _tc_transform_chunk(embeds[c], W, b2, out2d,
                                    offs[c], SPLITS[c])
    return out2d.reshape(TL, BATCH, DIM).transpose(1, 0, 2)


# speedup vs baseline: 2.9577x; 1.0336x over previous
import functools

import jax
import jax.numpy as jnp
from jax import lax
from jax.experimental import pallas as pl
from jax.experimental.pallas import tpu as pltpu
from jax.experimental.pallas import tpu_sc as plsc

BATCH = 4096
TL = 20
VOCAB = 100000
DIM = 768
NTOK = BATCH * TL

NUM_CORES = 2
NUM_SUBCORES = 16
NW = NUM_CORES * NUM_SUBCORES
UNIT = 4096
SCHUNK = 64

ROWS_BLK = 1024

SPLITS = (2, 5, 5, 5, 3)


def _sc_gather_chunk(table, idx_c, units):
    n_rows = units * UNIT
    b_per_w = n_rows // NW
    mesh = plsc.VectorSubcoreMesh(
        core_axis_name="c", subcore_axis_name="s",
        num_cores=NUM_CORES, num_subcores=NUM_SUBCORES)

    @functools.partial(
        pl.kernel,
        out_type=jax.ShapeDtypeStruct((n_rows, DIM), jnp.float32),
        mesh=mesh,
        compiler_params=pltpu.CompilerParams(use_tc_tiling_on_sc=True),
        scratch_types=[
            pltpu.VMEM((b_per_w,), jnp.int32),
            pltpu.VMEM((SCHUNK, DIM), jnp.float32),
            pltpu.VMEM((SCHUNK, DIM), jnp.float32),
            pltpu.SemaphoreType.DMA,
            pltpu.SemaphoreType.DMA,
        ],
    )
    def gather_kernel(table_hbm, idx_hbm, out_hbm, idx_v, rows_a, rows_b,
                      gsem, wsem):
        wid = lax.axis_index("s") * NUM_CORES + lax.axis_index("c")
        base = wid * b_per_w
        pltpu.sync_copy(idx_hbm.at[pl.ds(base, b_per_w)], idx_v)
        bufs = (rows_a, rows_b)
        nstream = b_per_w // SCHUNK
        gcp = [None] * nstream
        wcp = [None] * nstream
        gcp[0] = pltpu.async_copy(
            table_hbm.at[idx_v.at[pl.ds(0, SCHUNK)]], rows_a, gsem)
        for s in range(nstream):
            cur = bufs[s % 2]
            gcp[s].wait()
            if s >= 1:
                wcp[s - 1].wait()
            if s + 1 < nstream:
                gcp[s + 1] = pltpu.async_copy(
                    table_hbm.at[idx_v.at[pl.ds((s + 1) * SCHUNK, SCHUNK)]],
                    bufs[(s + 1) % 2], gsem)
            wcp[s] = pltpu.async_copy(
                cur, out_hbm.at[pl.ds(base + s * SCHUNK, SCHUNK)], wsem)
        wcp[nstream - 1].wait()

    return gather_kernel(table, idx_c)


def _mm_body(x_ref, w_ref, b_ref, o_ref):
    x = x_ref[...].astype(jnp.bfloat16)
    w = w_ref[...].astype(jnp.bfloat16)
    acc = lax.dot_general(x, w, (((1,), (1,)), ((), ())),
                          preferred_element_type=jnp.float32)
    o_ref[...] = acc + b_ref[...]


def _mm_body_alias(x_ref, w_ref, b_ref, prev_ref, o_ref):
    del prev_ref
    _mm_body(x_ref, w_ref, b_ref, o_ref)


def _tc_transform_chunk(x, W2, b2, prev, row0, units):
    blocks = units * UNIT // ROWS_BLK
    blk0 = row0 // ROWS_BLK
    out_map = functools.partial(lambda b0, i: (b0 + i, 0), blk0)
    x_spec = pl.BlockSpec((ROWS_BLK, DIM), lambda i: (i, 0))
    w_spec = pl.BlockSpec((DIM, DIM), lambda i: (0, 0))
    b_spec = pl.BlockSpec((1, DIM), lambda i: (0, 0))
    if prev is None:
        return pl.pallas_call(
            _mm_body,
            grid=(blocks,),
            in_specs=[x_spec, w_spec, b_spec],
            out_specs=pl.BlockSpec((ROWS_BLK, DIM), out_map),
            out_shape=jax.ShapeDtypeStruct((NTOK, DIM), jnp.float32),
        )(x, W2, b2)
    return pl.pallas_call(
        _mm_body_alias,
        grid=(blocks,),
        in_specs=[x_spec, w_spec, b_spec,
                  pl.BlockSpec(memory_space=pl.ANY)],
        out_specs=pl.BlockSpec((ROWS_BLK, DIM), out_map),
        out_shape=jax.ShapeDtypeStruct((NTOK, DIM), jnp.float32),
        input_output_aliases={3: 0},
    )(x, W2, b2, prev)


def kernel(token_ids, joint_embed, W, b):
    idx = token_ids.T.reshape(-1)
    b2 = b.reshape(1, DIM)
    offs = [0]
    for u in SPLITS:
        offs.append(offs[-1] + u * UNIT)
    embeds = [
        _sc_gather_chunk(joint_embed,
                         lax.slice(idx, (offs[c],), (offs[c + 1],)),
                         SPLITS[c])
        for c in range(len(SPLITS))
    ]
    out2d = None
    for c in range(len(SPLITS)):
        out2d = _tc_transform_chunk(embeds[c], W, b2, out2d,
                                    offs[c], SPLITS[c])
    return out2d.reshape(TL, BATCH, DIM).transpose(1, 0, 2)
